# baseline (speedup 1.0000x reference)
import functools

import jax
import jax.numpy as jnp
from jax import lax
from jax.experimental import pallas as pl
from jax.experimental.pallas import tpu as pltpu
from jax.experimental.pallas import tpu_sc as plsc

VOCAB = 100000
EMB = 32
CTX = 20
B = 1024

NC, NS = 2, 16
NW = NC * NS
N_IDX = B * CTX
CHUNK = 128
N_CHUNK = N_IDX // (NW * CHUNK)

V_BLK = 3072


N_ROW = NW and (N_IDX // NW)
N_GRP = N_ROW // 16


def _sc_gather(x_flat, emb4):
    mesh = plsc.VectorSubcoreMesh(core_axis_name="c", subcore_axis_name="s")

    @functools.partial(
        pl.kernel,
        out_type=jax.ShapeDtypeStruct((NW, N_CHUNK, CHUNK, EMB), jnp.float32),
        mesh=mesh,
        scratch_types=[
            pltpu.VMEM((N_CHUNK, CHUNK), jnp.int32),
            pltpu.VMEM((N_CHUNK, CHUNK), jnp.int32),
            pltpu.VMEM((N_ROW, 128), jnp.float32),
            pltpu.VMEM((N_CHUNK, CHUNK, EMB), jnp.float32),
            pltpu.SemaphoreType.DMA,
        ],
        compiler_params=pltpu.CompilerParams(use_tc_tiling_on_sc=False, needs_layout_passes=False),
    )
    def gather_kernel(idx_hbm, table_hbm, out_hbm, idx_v, sup_v, wide_v,
                      rows_v, sem):
        wid = lax.axis_index("s") * NC + lax.axis_index("c")
        pltpu.sync_copy(idx_hbm.at[wid], idx_v)
        for j in range(N_CHUNK):
            for t in range(CHUNK // 16):
                sl = pl.ds(t * 16, 16)
                sup_v[j, sl] = idx_v[j, sl] >> 2
        copies = [
            pltpu.async_copy(
                table_hbm.at[sup_v.at[j]],
                wide_v.at[pl.ds(j * CHUNK, CHUNK)],
                sem,
            )
            for j in range(N_CHUNK)
        ]
        for c in copies:
            c.wait()

        for j in range(N_CHUNK):
            def grp_body(t, _, j=j):
                idx16 = idx_v[j, pl.ds(t * 16, 16)]
                off = (idx16 & 3) * EMB
                ivec = lax.iota(jnp.int32, 16) + (j * CHUNK + t * 16)
                wvec = lax.iota(jnp.int32, 16) + t * 16
                jvec = jnp.full((16,), j, jnp.int32)
                for jj in range(EMB):
                    vals = plsc.load_gather(wide_v, [ivec, off + jj])
                    plsc.store_scatter(
                        rows_v,
                        [jvec, wvec, jnp.full((16,), jj, jnp.int32)],
                        vals,
                    )
                return 0

            lax.fori_loop(0, CHUNK // 16, grp_body, 0)
        pltpu.sync_copy(rows_v, out_hbm.at[wid])

    return gather_kernel(x_flat, emb4)


def _tc_matmul_kernel(e_ref, w_ref, b_ref, out_ref):
    e16 = e_ref[...].astype(jnp.bfloat16)
    w16 = w_ref[...].astype(jnp.bfloat16)
    acc = lax.dot_general(
        w16, e16,
        dimension_numbers=(((1,), (1,)), ((), ())),
        preferred_element_type=jnp.float32,
    )
    bias = lax.broadcast_in_dim(b_ref[...], (V_BLK, B), (0,))
    out_ref[...] = acc + bias


def _tc_matmul_t(e, W, b2d):
    grid = (pl.cdiv(VOCAB, V_BLK),)
    return pl.pallas_call(
        _tc_matmul_kernel,
        grid=grid,
        in_specs=[
            pl.BlockSpec((B, CTX * EMB), lambda i: (0, 0)),
            pl.BlockSpec((V_BLK, CTX * EMB), lambda i: (i, 0)),
            pl.BlockSpec((V_BLK,), lambda i: (i,)),
        ],
        out_specs=pl.BlockSpec((V_BLK, B), lambda i: (i, 0)),
        out_shape=jax.ShapeDtypeStruct((VOCAB, B), jnp.float32),
        compiler_params=pltpu.CompilerParams(vmem_limit_bytes=100 * 1024 * 1024),
    )(e, W, b2d)


def kernel(x, emb, W, b):
    x_flat = x.reshape(NW, N_CHUNK, CHUNK)
    emb4 = emb.reshape(VOCAB // 4, 4 * EMB)
    rows = _sc_gather(x_flat, emb4)
    e = rows.reshape(B, CTX * EMB)
    out_t = _tc_matmul_t(e, W, b)
    return out_t.T


# speedup vs baseline: 1.0943x; 1.0943x over previous
import functools

import jax
import jax.numpy as jnp
from jax import lax
from jax.experimental import pallas as pl
from jax.experimental.pallas import tpu as pltpu
from jax.experimental.pallas import tpu_sc as plsc

VOCAB = 100000
EMB = 32
CTX = 20
B = 1024

NC, NS = 2, 16
NW = NC * NS
N_IDX = B * CTX
CHUNK = 128
N_CHUNK = N_IDX // (NW * CHUNK)

V_BLK = 3072


def _sc_gather(x_flat, emb):
    mesh = plsc.VectorSubcoreMesh(core_axis_name="c", subcore_axis_name="s")

    @functools.partial(
        pl.kernel,
        out_type=jax.ShapeDtypeStruct((NW, N_CHUNK, CHUNK, EMB), jnp.float32),
        mesh=mesh,
        scratch_types=[
            pltpu.VMEM((N_CHUNK, CHUNK), jnp.int32),
            pltpu.VMEM((N_CHUNK, CHUNK, EMB), jnp.float32),
            pltpu.SemaphoreType.DMA,
        ],
        compiler_params=pltpu.CompilerParams(use_tc_tiling_on_sc=False),
    )
    def gather_kernel(idx_hbm, table_hbm, out_hbm, idx_v, rows_v, sem):
        wid = lax.axis_index("s") * NC + lax.axis_index("c")
        pltpu.sync_copy(idx_hbm.at[wid], idx_v)
        copies = [
            pltpu.async_copy(table_hbm.at[idx_v.at[j]], rows_v.at[j], sem)
            for j in range(N_CHUNK)
        ]
        for c in copies:
            c.wait()
        pltpu.sync_copy(rows_v, out_hbm.at[wid])

    return gather_kernel(x_flat, emb)


def _tc_matmul_kernel(e_ref, w_ref, b_ref, out_ref):
    e16 = e_ref[...].astype(jnp.bfloat16)
    w16 = w_ref[...].astype(jnp.bfloat16)
    acc = lax.dot_general(
        w16, e16,
        dimension_numbers=(((1,), (1,)), ((), ())),
        preferred_element_type=jnp.float32,
    )
    bias = lax.broadcast_in_dim(b_ref[...], (V_BLK, B), (0,))
    out_ref[...] = acc + bias


def _tc_matmul_t(e, W, b2d):
    grid = (pl.cdiv(VOCAB, V_BLK),)
    return pl.pallas_call(
        _tc_matmul_kernel,
        grid=grid,
        in_specs=[
            pl.BlockSpec((B, CTX * EMB), lambda i: (0, 0)),
            pl.BlockSpec((V_BLK, CTX * EMB), lambda i: (i, 0)),
            pl.BlockSpec((V_BLK,), lambda i: (i,)),
        ],
        out_specs=pl.BlockSpec((V_BLK, B), lambda i: (i, 0)),
        out_shape=jax.ShapeDtypeStruct((VOCAB, B), jnp.float32),
        compiler_params=pltpu.CompilerParams(vmem_limit_bytes=100 * 1024 * 1024),
    )(e, W, b2d)


def kernel(x, emb, W, b):
    x_flat = x.reshape(NW, N_CHUNK, CHUNK)
    rows = _sc_gather(x_flat, emb)
    e = rows.reshape(B, CTX * EMB)
    out_t = _tc_matmul_t(e, W, b)
    return out_t.T
